# baseline (speedup 1.0000x reference)
import functools

import jax
import jax.numpy as jnp
from jax import lax
from jax.experimental import pallas as pl
from jax.experimental.pallas import tpu as pltpu
from jax.experimental.pallas import tpu_sc as plsc

N = 10000
E = 320000
CH = 128
L = 16
NCH = CH // L
NC, NS = 2, 16
NW = NC * NS
EPT = E // NW
EPT_P = 10240
PADE = EPT_P - EPT
BE = 80
NB = EPT_P // BE
SB = 16
NSB = NB // SB
NBUF = 4
STRIPE = 632
NPAD = NS * STRIPE

_ROW_BLK = 2000


def _mm_body(x_ref, d_ref, w_ref, y_ref):
    y_ref[...] = jnp.dot(x_ref[...] * d_ref[...], w_ref[...],
                         preferred_element_type=jnp.float32).astype(jnp.bfloat16)


def _matmul(x, deg2, weight):
    return pl.pallas_call(
        _mm_body,
        grid=(N // _ROW_BLK,),
        in_specs=[
            pl.BlockSpec((_ROW_BLK, CH), lambda i: (i, 0)),
            pl.BlockSpec((_ROW_BLK, 1), lambda i: (i, 0)),
            pl.BlockSpec((CH, CH), lambda i: (0, 0)),
        ],
        out_specs=pl.BlockSpec((_ROW_BLK, CH), lambda i: (i, 0)),
        out_shape=jax.ShapeDtypeStruct((N, CH), jnp.bfloat16),
    )(x, deg2, weight)


def _ep_body(p_ref, d_ref, b_ref, o_ref):
    o_ref[...] = d_ref[...] * (p_ref[0] + p_ref[1]) + b_ref[...]


def _epilogue(partials, deg2, bias2):
    return pl.pallas_call(
        _ep_body,
        grid=(N // _ROW_BLK,),
        in_specs=[
            pl.BlockSpec((NC, _ROW_BLK, CH), lambda i: (0, i, 0)),
            pl.BlockSpec((_ROW_BLK, 1), lambda i: (i, 0)),
            pl.BlockSpec((1, CH), lambda i: (0, 0)),
        ],
        out_specs=pl.BlockSpec((_ROW_BLK, CH), lambda i: (i, 0)),
        out_shape=jax.ShapeDtypeStruct((N, CH), jnp.float32),
    )(partials, deg2, bias2)


def _agg_body(y_hbm, rec_hbm, out_hbm, recs, rows, frows, acc,
              gsem0, gsem1, gsem2, gsem3, isem0, isem1):
    c = lax.axis_index("c")
    s = lax.axis_index("s")
    wid = c * NS + s

    zv = jnp.zeros((L,), jnp.float32)

    def _zrow(r, carry):
        for g in range(NCH):
            frows[r, pl.ds(g * L, L)] = zv
        return carry
    lax.fori_loop(0, BE, _zrow, 0)

    base = s * STRIPE
    for r in range(STRIPE // BE):
        pltpu.sync_copy(frows, acc.at[pl.ds(base + r * BE, BE)])
    pltpu.sync_copy(frows.at[pl.ds(0, STRIPE % BE)],
                    acc.at[pl.ds(base + (STRIPE // BE) * BE, STRIPE % BE)])
    plsc.subcore_barrier()

    isems = (isem0, isem1)
    gsems = (gsem0, gsem1, gsem2, gsem3)

    himask = jnp.full((L,), -65536, jnp.int32)

    def _block(jin, par, b, refill):
        pltpu.make_async_copy(y_hbm.at[recs.at[par, jin, 0]],
                              rows.at[b], gsems[b]).wait()

        def _widen(e, carry):
            for g in range(CH // (2 * L)):
                w = rows[b, e, pl.ds(g * L, L)]
                lo = plsc.bitcast(w << 16, jnp.float32)
                hi = plsc.bitcast(w & himask, jnp.float32)
                frows[e, pl.ds(g * 2 * L, L)] = lo
                frows[e, pl.ds(g * 2 * L + L, L)] = hi
            return carry
        lax.fori_loop(0, BE, _widen, 0)

        pltpu.sync_copy(frows, acc.at[recs.at[par, jin, 1]], add=True)
        if refill:
            pltpu.async_copy(y_hbm.at[recs.at[par, jin + NBUF, 0]],
                             rows.at[b], gsems[b])

    def _superblock(S, par):
        def _quad(i4, carry):
            for u in range(NBUF):
                _block(i4 * NBUF + u, par, u, refill=True)
            return carry

        lax.fori_loop(0, SB // NBUF - 1, _quad, 0)
        for u in range(NBUF):
            _block(SB - NBUF + u, par, u, refill=False)

        opar = 1 - par

        @pl.when(S + 1 < NSB)
        def _():
            pltpu.make_async_copy(rec_hbm.at[wid, pl.ds((S + 1) * SB, SB)],
                                  recs.at[opar], isems[opar]).wait()
            for u in range(NBUF):
                pltpu.async_copy(y_hbm.at[recs.at[opar, u, 0]],
                                 rows.at[u], gsems[u])

        @pl.when(S + 2 < NSB)
        def _():
            pltpu.async_copy(rec_hbm.at[wid, pl.ds((S + 2) * SB, SB)],
                             recs.at[par], isems[par])

    pltpu.sync_copy(rec_hbm.at[wid, pl.ds(0, SB)], recs.at[0])
    for u in range(NBUF):
        pltpu.async_copy(y_hbm.at[recs.at[0, u, 0]], rows.at[u], gsems[u])
    pltpu.async_copy(rec_hbm.at[wid, pl.ds(SB, SB)], recs.at[1], isem1)

    def _outer(ss, carry):
        _superblock(ss * 2, 0)
        _superblock(ss * 2 + 1, 1)
        return carry

    lax.fori_loop(0, NSB // 2, _outer, 0)

    plsc.subcore_barrier()
    pltpu.sync_copy(acc.at[pl.ds(base, STRIPE)],
                    out_hbm.at[c, pl.ds(base, STRIPE)])


@functools.cache
def _agg():
    return functools.partial(
        pl.kernel,
        out_type=jax.ShapeDtypeStruct((NC, NPAD, CH), jnp.float32),
        mesh=plsc.VectorSubcoreMesh(core_axis_name="c", subcore_axis_name="s",
                                    num_cores=NC, num_subcores=NS),
        scratch_types=[
            pltpu.VMEM((2, SB, 2, BE), jnp.int32),
            pltpu.VMEM((NBUF, BE, CH // 2), jnp.int32),
            pltpu.VMEM((BE, CH), jnp.float32),
            pltpu.VMEM_SHARED((NPAD, CH), jnp.float32),
            pltpu.SemaphoreType.DMA,
            pltpu.SemaphoreType.DMA,
            pltpu.SemaphoreType.DMA,
            pltpu.SemaphoreType.DMA,
            pltpu.SemaphoreType.DMA,
            pltpu.SemaphoreType.DMA,
        ],
        compiler_params=pltpu.CompilerParams(needs_layout_passes=False,
                                             use_tc_tiling_on_sc=False),
    )(_agg_body)


_PERM = sum([[g * 32 + i, g * 32 + 16 + i] for g in range(CH // 32)
             for i in range(16)], [])


def kernel(x, weight, bias, degrees, src_index, dst_index):
    deg2 = degrees.reshape(N, 1)
    wp = weight[:, jnp.array(_PERM, dtype=jnp.int32)]
    y_bf = _matmul(x, deg2, wp)
    y = jax.lax.bitcast_convert_type(y_bf.reshape(N, CH // 2, 2),
                                     jnp.int32)
    srcp = jnp.pad(src_index.astype(jnp.int32).reshape(NW, EPT),
                   ((0, 0), (0, PADE)))
    dstp = jnp.pad(dst_index.astype(jnp.int32).reshape(NW, EPT),
                   ((0, 0), (0, PADE)), constant_values=NPAD - 1)
    rec = jnp.stack([srcp.reshape(NW, NB, BE), dstp.reshape(NW, NB, BE)], axis=2)
    partials = _agg()(y, rec)
    return _epilogue(partials, deg2, bias.reshape(1, CH))


# speedup vs baseline: 1.0882x; 1.0882x over previous
import functools

import jax
import jax.numpy as jnp
from jax import lax
from jax.experimental import pallas as pl
from jax.experimental.pallas import tpu as pltpu
from jax.experimental.pallas import tpu_sc as plsc

N = 10000
E = 320000
CH = 128
L = 16
NCH = CH // L
NC, NS = 2, 16
NW = NC * NS
EPT = E // NW
EPT_P = 10240
PADE = EPT_P - EPT
BE = 128
NB = EPT_P // BE
SB = 8
NSB = NB // SB
STRIPE = 632
NPAD = NS * STRIPE

_ROW_BLK = 2000


def _mm_body(x_ref, d_ref, w_ref, y_ref):
    y_ref[...] = jnp.dot(x_ref[...] * d_ref[...], w_ref[...],
                         preferred_element_type=jnp.float32)


def _matmul(x, deg2, weight):
    return pl.pallas_call(
        _mm_body,
        grid=(N // _ROW_BLK,),
        in_specs=[
            pl.BlockSpec((_ROW_BLK, CH), lambda i: (i, 0)),
            pl.BlockSpec((_ROW_BLK, 1), lambda i: (i, 0)),
            pl.BlockSpec((CH, CH), lambda i: (0, 0)),
        ],
        out_specs=pl.BlockSpec((_ROW_BLK, CH), lambda i: (i, 0)),
        out_shape=jax.ShapeDtypeStruct((N, CH), jnp.float32),
    )(x, deg2, weight)


def _ep_body(p_ref, d_ref, b_ref, o_ref):
    o_ref[...] = d_ref[...] * (p_ref[0] + p_ref[1]) + b_ref[...]


def _epilogue(partials, deg2, bias2):
    return pl.pallas_call(
        _ep_body,
        grid=(N // _ROW_BLK,),
        in_specs=[
            pl.BlockSpec((NC, _ROW_BLK, CH), lambda i: (0, i, 0)),
            pl.BlockSpec((_ROW_BLK, 1), lambda i: (i, 0)),
            pl.BlockSpec((1, CH), lambda i: (0, 0)),
        ],
        out_specs=pl.BlockSpec((_ROW_BLK, CH), lambda i: (i, 0)),
        out_shape=jax.ShapeDtypeStruct((N, CH), jnp.float32),
    )(partials, deg2, bias2)


def _agg_body(y_hbm, rec_hbm, out_hbm, recs, rows, acc,
              gsem0, gsem1, isem0, isem1):
    c = lax.axis_index("c")
    s = lax.axis_index("s")
    wid = c * NS + s

    zv = jnp.zeros((L,), jnp.float32)

    def _zrow(r, carry):
        for g in range(NCH):
            rows[0, r, pl.ds(g * L, L)] = zv
        return carry
    lax.fori_loop(0, BE, _zrow, 0)

    base = s * STRIPE
    for r in range(STRIPE // BE):
        pltpu.sync_copy(rows.at[0], acc.at[pl.ds(base + r * BE, BE)])
    pltpu.sync_copy(rows.at[0, pl.ds(0, STRIPE % BE)],
                    acc.at[pl.ds(base + (STRIPE // BE) * BE, STRIPE % BE)])
    plsc.subcore_barrier()

    isems = (isem0, isem1)

    def _block(jin, par, b, sem, refill):
        pltpu.make_async_copy(y_hbm.at[recs.at[par, jin, 0]],
                              rows.at[b], sem).wait()
        pltpu.sync_copy(rows.at[b], acc.at[recs.at[par, jin, 1]], add=True)
        if refill:
            pltpu.async_copy(y_hbm.at[recs.at[par, jin + 2, 0]],
                             rows.at[b], sem)

    def _superblock(S, par):
        def _pair(i2, carry):
            _block(i2 * 2, par, 0, gsem0, refill=True)
            _block(i2 * 2 + 1, par, 1, gsem1, refill=True)
            return carry

        lax.fori_loop(0, SB // 2 - 1, _pair, 0)
        _block(SB - 2, par, 0, gsem0, refill=False)
        _block(SB - 1, par, 1, gsem1, refill=False)

        opar = 1 - par

        @pl.when(S + 1 < NSB)
        def _():
            pltpu.make_async_copy(rec_hbm.at[wid, pl.ds((S + 1) * SB, SB)],
                                  recs.at[opar], isems[opar]).wait()
            pltpu.async_copy(y_hbm.at[recs.at[opar, 0, 0]], rows.at[0], gsem0)
            pltpu.async_copy(y_hbm.at[recs.at[opar, 1, 0]], rows.at[1], gsem1)

        @pl.when(S + 2 < NSB)
        def _():
            pltpu.async_copy(rec_hbm.at[wid, pl.ds((S + 2) * SB, SB)],
                             recs.at[par], isems[par])

    pltpu.sync_copy(rec_hbm.at[wid, pl.ds(0, SB)], recs.at[0])
    pltpu.async_copy(y_hbm.at[recs.at[0, 0, 0]], rows.at[0], gsem0)
    pltpu.async_copy(y_hbm.at[recs.at[0, 1, 0]], rows.at[1], gsem1)
    pltpu.async_copy(rec_hbm.at[wid, pl.ds(SB, SB)], recs.at[1], isem1)

    def _outer(ss, carry):
        _superblock(ss * 2, 0)
        _superblock(ss * 2 + 1, 1)
        return carry

    lax.fori_loop(0, NSB // 2, _outer, 0)

    plsc.subcore_barrier()
    pltpu.sync_copy(acc.at[pl.ds(base, STRIPE)],
                    out_hbm.at[c, pl.ds(base, STRIPE)])


@functools.cache
def _agg():
    return functools.partial(
        pl.kernel,
        out_type=jax.ShapeDtypeStruct((NC, NPAD, CH), jnp.float32),
        mesh=plsc.VectorSubcoreMesh(core_axis_name="c", subcore_axis_name="s",
                                    num_cores=NC, num_subcores=NS),
        scratch_types=[
            pltpu.VMEM((2, SB, 2, BE), jnp.int32),
            pltpu.VMEM((2, BE, CH), jnp.float32),
            pltpu.VMEM_SHARED((NPAD, CH), jnp.float32),
            pltpu.SemaphoreType.DMA,
            pltpu.SemaphoreType.DMA,
            pltpu.SemaphoreType.DMA,
            pltpu.SemaphoreType.DMA,
        ],
        compiler_params=pltpu.CompilerParams(needs_layout_passes=False,
                                             use_tc_tiling_on_sc=False),
    )(_agg_body)


def kernel(x, weight, bias, degrees, src_index, dst_index):
    deg2 = degrees.reshape(N, 1)
    y = _matmul(x, deg2, weight)
    srcp = jnp.pad(src_index.astype(jnp.int32).reshape(NW, EPT),
                   ((0, 0), (0, PADE)))
    dstp = jnp.pad(dst_index.astype(jnp.int32).reshape(NW, EPT),
                   ((0, 0), (0, PADE)), constant_values=NPAD - 1)
    rec = jnp.stack([srcp.reshape(NW, NB, BE), dstp.reshape(NW, NB, BE)], axis=2)
    partials = _agg()(y, rec)
    return _epilogue(partials, deg2, bias.reshape(1, CH))
